# baseline (speedup 1.0000x reference)
import functools

import jax
import jax.numpy as jnp
from jax.experimental import pallas as pl
from jax.experimental.pallas import tpu as pltpu

_VOCAB_TILE = 1024


def _gather_body(ids_ref, emb_ref, out_ref):
    out_ref[...] = emb_ref[...]


def _matmul_body(x_ref, w_ref, out_ref):
    x = x_ref[...]
    w = w_ref[...]
    out_ref[...] = jax.lax.dot_general(
        x, w,
        dimension_numbers=(((1,), (1,)), ((), ())),
        preferred_element_type=jnp.float32,
    )


def kernel(input_ids, embed_weight, lm_head_weight):
    batch, seq = input_ids.shape
    n_tok = batch * seq
    vocab, hidden = embed_weight.shape
    ids = input_ids.reshape(n_tok).astype(jnp.int32)

    emb3 = embed_weight.reshape(vocab, 1, hidden)
    x = pl.pallas_call(
        _gather_body,
        grid_spec=pltpu.PrefetchScalarGridSpec(
            num_scalar_prefetch=1,
            grid=(n_tok,),
            in_specs=[
                pl.BlockSpec((1, 1, hidden), lambda i, ids_ref: (ids_ref[i], 0, 0)),
            ],
            out_specs=pl.BlockSpec((1, 1, hidden), lambda i, ids_ref: (i, 0, 0)),
        ),
        out_shape=jax.ShapeDtypeStruct((n_tok, 1, hidden), jnp.float32),
    )(ids, emb3)
    x = x.reshape(n_tok, hidden)

    n_tiles = pl.cdiv(vocab, _VOCAB_TILE)
    logits = pl.pallas_call(
        _matmul_body,
        grid=(n_tiles,),
        in_specs=[
            pl.BlockSpec((n_tok, hidden), lambda j: (0, 0)),
            pl.BlockSpec((_VOCAB_TILE, hidden), lambda j: (j, 0)),
        ],
        out_specs=pl.BlockSpec((n_tok, _VOCAB_TILE), lambda j: (0, j)),
        out_shape=jax.ShapeDtypeStruct((n_tok, vocab), jnp.float32),
    )(x, lm_head_weight)

    return logits.reshape(batch, seq, vocab)


# speedup vs baseline: 1.3449x; 1.3449x over previous
import jax
import jax.numpy as jnp
from jax.experimental import pallas as pl
from jax.experimental.pallas import tpu as pltpu

_VOCAB_TILE = 1024


def _body(ids_ref, emb_hbm, w_ref, out_ref, x_ref, sem):
    n_tok = x_ref.shape[0]

    @pl.when(pl.program_id(0) == 0)
    def _gather():
        def issue(i, c):
            pltpu.make_async_copy(
                emb_hbm.at[pl.ds(ids_ref[i], 1), :],
                x_ref.at[pl.ds(i, 1), :],
                sem,
            ).start()
            return c

        jax.lax.fori_loop(0, n_tok, issue, 0)
        pltpu.make_async_copy(
            emb_hbm.at[pl.ds(0, n_tok), :], x_ref, sem
        ).wait()

    out_ref[...] = jax.lax.dot_general(
        x_ref[...],
        w_ref[...],
        dimension_numbers=(((1,), (1,)), ((), ())),
        preferred_element_type=jnp.float32,
    )


def kernel(input_ids, embed_weight, lm_head_weight):
    batch, seq = input_ids.shape
    n_tok = batch * seq
    vocab, hidden = embed_weight.shape
    ids = input_ids.reshape(n_tok).astype(jnp.int32)

    n_tiles = pl.cdiv(vocab, _VOCAB_TILE)
    logits = pl.pallas_call(
        _body,
        grid_spec=pltpu.PrefetchScalarGridSpec(
            num_scalar_prefetch=1,
            grid=(n_tiles,),
            in_specs=[
                pl.BlockSpec(memory_space=pl.ANY),
                pl.BlockSpec((_VOCAB_TILE, hidden), lambda j, ids_ref: (j, 0)),
            ],
            out_specs=pl.BlockSpec((n_tok, _VOCAB_TILE), lambda j, ids_ref: (0, j)),
            scratch_shapes=[
                pltpu.VMEM((n_tok, hidden), jnp.float32),
                pltpu.SemaphoreType.DMA,
            ],
        ),
        out_shape=jax.ShapeDtypeStruct((n_tok, vocab), jnp.float32),
    )(ids, embed_weight, lm_head_weight)

    return logits.reshape(batch, seq, vocab)


# speedup vs baseline: 1.3460x; 1.0008x over previous
import jax
import jax.numpy as jnp
from jax.experimental import pallas as pl
from jax.experimental.pallas import tpu as pltpu

_VOCAB_TILE = 1024


def _body(ids_ref, emb_hbm, w_ref, out_ref, x_ref, xb_ref, sem):
    n_tok = x_ref.shape[0]

    @pl.when(pl.program_id(0) == 0)
    def _gather():
        def issue(i, c):
            pltpu.make_async_copy(
                emb_hbm.at[pl.ds(ids_ref[i], 1), :],
                x_ref.at[pl.ds(i, 1), :],
                sem,
            ).start()
            return c

        jax.lax.fori_loop(0, n_tok, issue, 0)
        pltpu.make_async_copy(
            emb_hbm.at[pl.ds(0, n_tok), :], x_ref, sem
        ).wait()
        xb_ref[...] = x_ref[...].astype(jnp.bfloat16)

    out_ref[...] = jax.lax.dot_general(
        xb_ref[...],
        w_ref[...].astype(jnp.bfloat16),
        dimension_numbers=(((1,), (1,)), ((), ())),
        preferred_element_type=jnp.float32,
    )


def kernel(input_ids, embed_weight, lm_head_weight):
    batch, seq = input_ids.shape
    n_tok = batch * seq
    vocab, hidden = embed_weight.shape
    ids = input_ids.reshape(n_tok).astype(jnp.int32)

    n_tiles = pl.cdiv(vocab, _VOCAB_TILE)
    logits = pl.pallas_call(
        _body,
        grid_spec=pltpu.PrefetchScalarGridSpec(
            num_scalar_prefetch=1,
            grid=(n_tiles,),
            in_specs=[
                pl.BlockSpec(memory_space=pl.ANY),
                pl.BlockSpec((_VOCAB_TILE, hidden), lambda j, ids_ref: (j, 0)),
            ],
            out_specs=pl.BlockSpec((n_tok, _VOCAB_TILE), lambda j, ids_ref: (0, j)),
            scratch_shapes=[
                pltpu.VMEM((n_tok, hidden), jnp.float32),
                pltpu.VMEM((n_tok, hidden), jnp.bfloat16),
                pltpu.SemaphoreType.DMA,
            ],
        ),
        out_shape=jax.ShapeDtypeStruct((n_tok, vocab), jnp.float32),
    )(ids, embed_weight, lm_head_weight)

    return logits.reshape(batch, seq, vocab)


# speedup vs baseline: 6.2407x; 4.6366x over previous
import jax
import jax.numpy as jnp
from jax.experimental import pallas as pl
from jax.experimental.pallas import tpu as pltpu

_VOCAB_TILE = 1024


def _body(ids_ref, emb_hbm, w_ref, out_ref, x_ref, xb_ref, sem):
    n_tok = x_ref.shape[0]
    seq = out_ref.shape[0]
    batch = out_ref.shape[2]

    @pl.when(pl.program_id(0) == 0)
    def _gather():
        def issue(i, c):
            pltpu.make_async_copy(
                emb_hbm.at[pl.ds(ids_ref[i], 1), :],
                x_ref.at[pl.ds(i, 1), :],
                sem,
            ).start()
            return c

        jax.lax.fori_loop(0, n_tok, issue, 0)
        pltpu.make_async_copy(
            emb_hbm.at[pl.ds(0, n_tok), :], x_ref, sem
        ).wait()
        xb_ref[...] = x_ref[...].astype(jnp.bfloat16)

    w = w_ref[...].astype(jnp.bfloat16)
    for s in range(seq):
        xs = xb_ref[s * batch : (s + 1) * batch, :]
        out_ref[s, :, :] = jax.lax.dot_general(
            w,
            xs,
            dimension_numbers=(((1,), (1,)), ((), ())),
            preferred_element_type=jnp.float32,
        )


def kernel(input_ids, embed_weight, lm_head_weight):
    batch, seq = input_ids.shape
    n_tok = batch * seq
    vocab, hidden = embed_weight.shape
    ids = input_ids.T.reshape(n_tok).astype(jnp.int32)

    n_tiles = pl.cdiv(vocab, _VOCAB_TILE)
    logits_svb = pl.pallas_call(
        _body,
        grid_spec=pltpu.PrefetchScalarGridSpec(
            num_scalar_prefetch=1,
            grid=(n_tiles,),
            in_specs=[
                pl.BlockSpec(memory_space=pl.ANY),
                pl.BlockSpec((_VOCAB_TILE, hidden), lambda j, ids_ref: (j, 0)),
            ],
            out_specs=pl.BlockSpec(
                (seq, _VOCAB_TILE, batch), lambda j, ids_ref: (0, j, 0)
            ),
            scratch_shapes=[
                pltpu.VMEM((n_tok, hidden), jnp.float32),
                pltpu.VMEM((n_tok, hidden), jnp.bfloat16),
                pltpu.SemaphoreType.DMA,
            ],
        ),
        out_shape=jax.ShapeDtypeStruct((seq, vocab, batch), jnp.float32),
    )(ids, embed_weight, lm_head_weight)

    return jnp.transpose(logits_svb, (2, 0, 1))


# speedup vs baseline: 8.1433x; 1.3049x over previous
import jax
import jax.numpy as jnp
from jax.experimental import pallas as pl
from jax.experimental.pallas import tpu as pltpu

_VOCAB_TILE = 2048


def _body(ids_ref, emb_hbm, w_ref, out_ref, x_ref, xb_ref, sem):
    n_tok = x_ref.shape[0]
    seq = out_ref.shape[0]
    batch = out_ref.shape[2]

    @pl.when(pl.program_id(0) == 0)
    def _gather():
        def issue(i, c):
            pltpu.make_async_copy(
                emb_hbm.at[pl.ds(ids_ref[i], 1), :],
                x_ref.at[pl.ds(i, 1), :],
                sem,
            ).start()
            return c

        jax.lax.fori_loop(0, n_tok, issue, 0)
        pltpu.make_async_copy(
            emb_hbm.at[pl.ds(0, n_tok), :], x_ref, sem
        ).wait()
        xb_ref[...] = x_ref[...].astype(jnp.bfloat16)

    w = w_ref[...].astype(jnp.bfloat16)
    for s in range(seq):
        xs = xb_ref[s * batch : (s + 1) * batch, :]
        out_ref[s, :, :] = jax.lax.dot_general(
            w,
            xs,
            dimension_numbers=(((1,), (1,)), ((), ())),
            preferred_element_type=jnp.float32,
        )


def kernel(input_ids, embed_weight, lm_head_weight):
    batch, seq = input_ids.shape
    n_tok = batch * seq
    vocab, hidden = embed_weight.shape
    ids = input_ids.T.reshape(n_tok).astype(jnp.int32)

    n_tiles = pl.cdiv(vocab, _VOCAB_TILE)
    logits_svb = pl.pallas_call(
        _body,
        grid_spec=pltpu.PrefetchScalarGridSpec(
            num_scalar_prefetch=1,
            grid=(n_tiles,),
            in_specs=[
                pl.BlockSpec(memory_space=pl.ANY),
                pl.BlockSpec((_VOCAB_TILE, hidden), lambda j, ids_ref: (j, 0)),
            ],
            out_specs=pl.BlockSpec(
                (seq, _VOCAB_TILE, batch), lambda j, ids_ref: (0, j, 0)
            ),
            scratch_shapes=[
                pltpu.VMEM((n_tok, hidden), jnp.float32),
                pltpu.VMEM((n_tok, hidden), jnp.bfloat16),
                pltpu.SemaphoreType.DMA,
            ],
        ),
        out_shape=jax.ShapeDtypeStruct((seq, vocab, batch), jnp.float32),
    )(ids, embed_weight, lm_head_weight)

    return jnp.transpose(logits_svb, (2, 0, 1))


# speedup vs baseline: 9.1524x; 1.1239x over previous
import jax
import jax.numpy as jnp
from jax.experimental import pallas as pl
from jax.experimental.pallas import tpu as pltpu

_VOCAB_TILE = 4096


def _body(ids_ref, emb_hbm, w_ref, out_ref, x_ref, xb_ref, sem):
    n_tok = x_ref.shape[0]
    seq = out_ref.shape[0]
    batch = out_ref.shape[2]

    @pl.when(pl.program_id(0) == 0)
    def _gather():
        def issue(i, c):
            pltpu.make_async_copy(
                emb_hbm.at[pl.ds(ids_ref[i], 1), :],
                x_ref.at[pl.ds(i, 1), :],
                sem,
            ).start()
            return c

        jax.lax.fori_loop(0, n_tok, issue, 0)
        pltpu.make_async_copy(
            emb_hbm.at[pl.ds(0, n_tok), :], x_ref, sem
        ).wait()
        xb_ref[...] = x_ref[...].astype(jnp.bfloat16)

    w = w_ref[...].astype(jnp.bfloat16)
    for s in range(seq):
        xs = xb_ref[s * batch : (s + 1) * batch, :]
        out_ref[s, :, :] = jax.lax.dot_general(
            w,
            xs,
            dimension_numbers=(((1,), (1,)), ((), ())),
            preferred_element_type=jnp.float32,
        )


def kernel(input_ids, embed_weight, lm_head_weight):
    batch, seq = input_ids.shape
    n_tok = batch * seq
    vocab, hidden = embed_weight.shape
    ids = input_ids.T.reshape(n_tok).astype(jnp.int32)

    n_tiles = pl.cdiv(vocab, _VOCAB_TILE)
    logits_svb = pl.pallas_call(
        _body,
        grid_spec=pltpu.PrefetchScalarGridSpec(
            num_scalar_prefetch=1,
            grid=(n_tiles,),
            in_specs=[
                pl.BlockSpec(memory_space=pl.ANY),
                pl.BlockSpec((_VOCAB_TILE, hidden), lambda j, ids_ref: (j, 0)),
            ],
            out_specs=pl.BlockSpec(
                (seq, _VOCAB_TILE, batch), lambda j, ids_ref: (0, j, 0)
            ),
            scratch_shapes=[
                pltpu.VMEM((n_tok, hidden), jnp.float32),
                pltpu.VMEM((n_tok, hidden), jnp.bfloat16),
                pltpu.SemaphoreType.DMA,
            ],
        ),
        out_shape=jax.ShapeDtypeStruct((seq, vocab, batch), jnp.float32),
    )(ids, embed_weight, lm_head_weight)

    return jnp.transpose(logits_svb, (2, 0, 1))


# speedup vs baseline: 9.1778x; 1.0028x over previous
import jax
import jax.numpy as jnp
from jax.experimental import pallas as pl
from jax.experimental.pallas import tpu as pltpu

_VOCAB_TILE = 8192


def _body(ids_ref, emb_hbm, w_ref, out_ref, x_ref, xb_ref, sem):
    n_tok = x_ref.shape[0]
    seq = out_ref.shape[0]
    batch = out_ref.shape[2]

    @pl.when(pl.program_id(0) == 0)
    def _gather():
        def issue(i, c):
            pltpu.make_async_copy(
                emb_hbm.at[pl.ds(ids_ref[i], 1), :],
                x_ref.at[pl.ds(i, 1), :],
                sem,
            ).start()
            return c

        jax.lax.fori_loop(0, n_tok, issue, 0)
        pltpu.make_async_copy(
            emb_hbm.at[pl.ds(0, n_tok), :], x_ref, sem
        ).wait()
        xb_ref[...] = x_ref[...].astype(jnp.bfloat16)

    w = w_ref[...].astype(jnp.bfloat16)
    for s in range(seq):
        xs = xb_ref[s * batch : (s + 1) * batch, :]
        out_ref[s, :, :] = jax.lax.dot_general(
            w,
            xs,
            dimension_numbers=(((1,), (1,)), ((), ())),
            preferred_element_type=jnp.float32,
        )


def kernel(input_ids, embed_weight, lm_head_weight):
    batch, seq = input_ids.shape
    n_tok = batch * seq
    vocab, hidden = embed_weight.shape
    ids = input_ids.T.reshape(n_tok).astype(jnp.int32)

    n_tiles = pl.cdiv(vocab, _VOCAB_TILE)
    logits_svb = pl.pallas_call(
        _body,
        grid_spec=pltpu.PrefetchScalarGridSpec(
            num_scalar_prefetch=1,
            grid=(n_tiles,),
            in_specs=[
                pl.BlockSpec(memory_space=pl.ANY),
                pl.BlockSpec((_VOCAB_TILE, hidden), lambda j, ids_ref: (j, 0)),
            ],
            out_specs=pl.BlockSpec(
                (seq, _VOCAB_TILE, batch), lambda j, ids_ref: (0, j, 0)
            ),
            scratch_shapes=[
                pltpu.VMEM((n_tok, hidden), jnp.float32),
                pltpu.VMEM((n_tok, hidden), jnp.bfloat16),
                pltpu.SemaphoreType.DMA,
            ],
        ),
        out_shape=jax.ShapeDtypeStruct((seq, vocab, batch), jnp.float32),
    )(ids, embed_weight, lm_head_weight)

    return jnp.transpose(logits_svb, (2, 0, 1))
